# baseline (speedup 1.0000x reference)
import jax
import jax.numpy as jnp
from jax.experimental import pallas as pl
from jax.experimental.pallas import tpu as pltpu

_BS = 2000
_NBUF = 8


def _mlp_kernel(sc_ref, hA_hbm, hB_hbm, w1_hbm, w2_hbm, wh_hbm, out_ref,
                abuf, bbuf, w1buf, w2buf, whbuf, in_sem, w_sem):
    i = pl.program_id(0)
    nb = pl.num_programs(0)

    def start(block, slot):
        pltpu.make_async_copy(
            hA_hbm.at[pl.ds(block * _BS, _BS), :], abuf.at[slot],
            in_sem.at[0, slot]).start()
        pltpu.make_async_copy(
            hB_hbm.at[pl.ds(block * _BS, _BS), :], bbuf.at[slot],
            in_sem.at[1, slot]).start()

    @pl.when(i == 0)
    def _prologue():
        for s in range(_NBUF):
            start(s, s)
        w1c = pltpu.make_async_copy(w1_hbm, w1buf, w_sem.at[0])
        w2c = pltpu.make_async_copy(w2_hbm, w2buf, w_sem.at[1])
        whc = pltpu.make_async_copy(wh_hbm, whbuf, w_sem.at[2])
        w1c.start(); w2c.start(); whc.start()
        w1c.wait(); w2c.wait(); whc.wait()

    slot = jax.lax.rem(i, _NBUF)
    pltpu.make_async_copy(
        hA_hbm.at[pl.ds(i * _BS, _BS), :], abuf.at[slot],
        in_sem.at[0, slot]).wait()
    pltpu.make_async_copy(
        hB_hbm.at[pl.ds(i * _BS, _BS), :], bbuf.at[slot],
        in_sem.at[1, slot]).wait()

    H = 128
    f = jnp.dot(abuf[slot], w1buf[0:H],
                preferred_element_type=jnp.float32)
    f = f + jnp.dot(bbuf[slot], w1buf[H:2 * H],
                    preferred_element_type=jnp.float32)
    f = f + jnp.dot(sc_ref[...], w1buf[2 * H:2 * H + 8],
                    preferred_element_type=jnp.float32)
    f = jnp.tanh(f + w1buf[2 * H + 8:2 * H + 9])
    f = jnp.tanh(jnp.dot(f, w2buf[0:H], preferred_element_type=jnp.float32)
                 + w2buf[H:H + 1])
    out_ref[...] = (jnp.dot(f, whbuf[0:64], preferred_element_type=jnp.float32)
                    + whbuf[64:65])

    @pl.when(i + _NBUF < nb)
    def _prefetch():
        start(i + _NBUF, slot)


def kernel(xi, h_A, h_B, E_val, I_val, L_val, q_val,
           W1, b1, W2, b2, Ww, bw, Wm, bm):
    B, H = h_A.shape
    D1 = W1.shape[1]
    D2 = W2.shape[1]

    zeros = jnp.zeros((B, 3), dtype=xi.dtype)
    scal = jnp.concatenate([xi, E_val, I_val, L_val, q_val, zeros], axis=-1)
    w1p = jnp.concatenate([
        W1[1:1 + H],
        W1[1 + H:1 + 2 * H],
        W1[0:1], W1[1 + 2 * H:], jnp.zeros((3, D1), W1.dtype),
        b1.reshape(1, D1),
    ], axis=0)
    w2p = jnp.concatenate([W2, b2.reshape(1, D2)], axis=0)
    whp = jnp.concatenate([
        jnp.concatenate([Ww, Wm], axis=1),
        jnp.concatenate([bw, bm]).reshape(1, 2),
    ], axis=0)

    grid = (B // _BS,)
    row = lambda i: (i, 0)

    out = pl.pallas_call(
        _mlp_kernel,
        grid=grid,
        in_specs=[
            pl.BlockSpec((_BS, 8), row),
            pl.BlockSpec(memory_space=pl.ANY),
            pl.BlockSpec(memory_space=pl.ANY),
            pl.BlockSpec(memory_space=pl.ANY),
            pl.BlockSpec(memory_space=pl.ANY),
            pl.BlockSpec(memory_space=pl.ANY),
        ],
        out_specs=pl.BlockSpec((_BS, 2), row),
        out_shape=jax.ShapeDtypeStruct((B, 2), jnp.float32),
        scratch_shapes=[
            pltpu.VMEM((_NBUF, _BS, H), jnp.float32),
            pltpu.VMEM((_NBUF, _BS, H), jnp.float32),
            pltpu.VMEM((2 * H + 9, D1), jnp.float32),
            pltpu.VMEM((H + 1, D2), jnp.float32),
            pltpu.VMEM((D2 + 1, 2), jnp.float32),
            pltpu.SemaphoreType.DMA((2, _NBUF)),
            pltpu.SemaphoreType.DMA((3,)),
        ],
        compiler_params=pltpu.CompilerParams(
            dimension_semantics=("arbitrary",),
            vmem_limit_bytes=100 * 1024 * 1024),
    )(scal, h_A, h_B, w1p, w2p, whp)

    return (out[:, 0:1], out[:, 1:2])


# speedup vs baseline: 1.0323x; 1.0323x over previous
import jax
import jax.numpy as jnp
from jax.experimental import pallas as pl
from jax.experimental.pallas import tpu as pltpu

_BS = 2000
_NBUF = 8


def _mlp_kernel(sc_hbm, hA_hbm, hB_hbm, w1_hbm, w2_hbm, wh_hbm, out_ref,
                sbuf, abuf, bbuf, w1buf, w2buf, whbuf, in_sem, w_sem):
    i = pl.program_id(0)
    nb = pl.num_programs(0)

    def start(block, slot):
        pltpu.make_async_copy(
            hA_hbm.at[pl.ds(block * _BS, _BS), :], abuf.at[slot],
            in_sem.at[0, slot]).start()
        pltpu.make_async_copy(
            hB_hbm.at[pl.ds(block * _BS, _BS), :], bbuf.at[slot],
            in_sem.at[1, slot]).start()
        pltpu.make_async_copy(
            sc_hbm.at[pl.ds(block * _BS, _BS), :], sbuf.at[slot],
            in_sem.at[2, slot]).start()

    @pl.when(i == 0)
    def _prologue():
        for s in range(_NBUF):
            start(s, s)
        w1c = pltpu.make_async_copy(w1_hbm, w1buf, w_sem.at[0])
        w2c = pltpu.make_async_copy(w2_hbm, w2buf, w_sem.at[1])
        whc = pltpu.make_async_copy(wh_hbm, whbuf, w_sem.at[2])
        w1c.start(); w2c.start(); whc.start()
        w1c.wait(); w2c.wait(); whc.wait()

    slot = jax.lax.rem(i, _NBUF)
    pltpu.make_async_copy(
        hA_hbm.at[pl.ds(i * _BS, _BS), :], abuf.at[slot],
        in_sem.at[0, slot]).wait()
    pltpu.make_async_copy(
        hB_hbm.at[pl.ds(i * _BS, _BS), :], bbuf.at[slot],
        in_sem.at[1, slot]).wait()
    pltpu.make_async_copy(
        sc_hbm.at[pl.ds(i * _BS, _BS), :], sbuf.at[slot],
        in_sem.at[2, slot]).wait()

    H = 128
    f = jnp.dot(abuf[slot], w1buf[0:H],
                preferred_element_type=jnp.float32)
    f = f + jnp.dot(bbuf[slot], w1buf[H:2 * H],
                    preferred_element_type=jnp.float32)
    f = f + jnp.dot(sbuf[slot], w1buf[2 * H:2 * H + 8],
                    preferred_element_type=jnp.float32)
    f = jnp.tanh(f + w1buf[2 * H + 8:2 * H + 9])
    f = jnp.tanh(jnp.dot(f, w2buf[0:H], preferred_element_type=jnp.float32)
                 + w2buf[H:H + 1])
    out_ref[...] = (jnp.dot(f, whbuf[0:64], preferred_element_type=jnp.float32)
                    + whbuf[64:65])

    @pl.when(i + _NBUF < nb)
    def _prefetch():
        start(i + _NBUF, slot)


def kernel(xi, h_A, h_B, E_val, I_val, L_val, q_val,
           W1, b1, W2, b2, Ww, bw, Wm, bm):
    B, H = h_A.shape
    D1 = W1.shape[1]
    D2 = W2.shape[1]

    zeros = jnp.zeros((B, 3), dtype=xi.dtype)
    scal = jnp.concatenate([xi, E_val, I_val, L_val, q_val, zeros], axis=-1)
    w1p = jnp.concatenate([
        W1[1:1 + H],
        W1[1 + H:1 + 2 * H],
        W1[0:1], W1[1 + 2 * H:], jnp.zeros((3, D1), W1.dtype),
        b1.reshape(1, D1),
    ], axis=0)
    w2p = jnp.concatenate([W2, b2.reshape(1, D2)], axis=0)
    whp = jnp.concatenate([
        jnp.concatenate([Ww, Wm], axis=1),
        jnp.concatenate([bw, bm]).reshape(1, 2),
    ], axis=0)

    grid = (B // _BS,)
    row = lambda i: (i, 0)

    out = pl.pallas_call(
        _mlp_kernel,
        grid=grid,
        in_specs=[
            pl.BlockSpec(memory_space=pl.ANY),
            pl.BlockSpec(memory_space=pl.ANY),
            pl.BlockSpec(memory_space=pl.ANY),
            pl.BlockSpec(memory_space=pl.ANY),
            pl.BlockSpec(memory_space=pl.ANY),
            pl.BlockSpec(memory_space=pl.ANY),
        ],
        out_specs=pl.BlockSpec((_BS, 2), row),
        out_shape=jax.ShapeDtypeStruct((B, 2), jnp.float32),
        scratch_shapes=[
            pltpu.VMEM((_NBUF, _BS, 8), jnp.float32),
            pltpu.VMEM((_NBUF, _BS, H), jnp.float32),
            pltpu.VMEM((_NBUF, _BS, H), jnp.float32),
            pltpu.VMEM((2 * H + 9, D1), jnp.float32),
            pltpu.VMEM((H + 1, D2), jnp.float32),
            pltpu.VMEM((D2 + 1, 2), jnp.float32),
            pltpu.SemaphoreType.DMA((3, _NBUF)),
            pltpu.SemaphoreType.DMA((3,)),
        ],
        compiler_params=pltpu.CompilerParams(
            dimension_semantics=("arbitrary",),
            vmem_limit_bytes=100 * 1024 * 1024),
    )(scal, h_A, h_B, w1p, w2p, whp)

    return (out[:, 0:1], out[:, 1:2])


# speedup vs baseline: 1.2058x; 1.1681x over previous
import jax
import jax.numpy as jnp
from jax.experimental import pallas as pl
from jax.experimental.pallas import tpu as pltpu

_BS = 3200
_NBUF = 8


def _mlp_kernel(sc_hbm, hA_hbm, hB_hbm, w1_hbm, w2_hbm, wh_hbm, out_ref,
                sbuf, abuf, bbuf, w1buf, w2buf, whbuf, in_sem, w_sem):
    i = pl.program_id(0)
    nb = pl.num_programs(0)

    def start(block, slot):
        pltpu.make_async_copy(
            hA_hbm.at[pl.ds(block * _BS, _BS), :], abuf.at[slot],
            in_sem.at[0, slot]).start()
        pltpu.make_async_copy(
            hB_hbm.at[pl.ds(block * _BS, _BS), :], bbuf.at[slot],
            in_sem.at[1, slot]).start()
        pltpu.make_async_copy(
            sc_hbm.at[:, pl.ds(block * _BS, _BS)], sbuf.at[slot],
            in_sem.at[2, slot]).start()

    @pl.when(i == 0)
    def _prologue():
        for s in range(_NBUF):
            start(s, s)
        w1c = pltpu.make_async_copy(w1_hbm, w1buf, w_sem.at[0])
        w2c = pltpu.make_async_copy(w2_hbm, w2buf, w_sem.at[1])
        whc = pltpu.make_async_copy(wh_hbm, whbuf, w_sem.at[2])
        w1c.start(); w2c.start(); whc.start()
        w1c.wait(); w2c.wait(); whc.wait()

    slot = jax.lax.rem(i, _NBUF)
    pltpu.make_async_copy(
        hA_hbm.at[pl.ds(i * _BS, _BS), :], abuf.at[slot],
        in_sem.at[0, slot]).wait()
    pltpu.make_async_copy(
        hB_hbm.at[pl.ds(i * _BS, _BS), :], bbuf.at[slot],
        in_sem.at[1, slot]).wait()
    pltpu.make_async_copy(
        sc_hbm.at[:, pl.ds(i * _BS, _BS)], sbuf.at[slot],
        in_sem.at[2, slot]).wait()

    H = 128
    f = jnp.dot(abuf[slot], w1buf[0:H],
                preferred_element_type=jnp.float32)
    f = f + jnp.dot(bbuf[slot], w1buf[H:2 * H],
                    preferred_element_type=jnp.float32)
    f = f + jax.lax.dot_general(
        sbuf[slot], w1buf[2 * H:2 * H + 5],
        (((0,), (0,)), ((), ())), preferred_element_type=jnp.float32)
    f = jnp.tanh(f + w1buf[2 * H + 5:2 * H + 6])
    f = jnp.tanh(jnp.dot(f, w2buf[0:H], preferred_element_type=jnp.float32)
                 + w2buf[H:H + 1])
    out_ref[...] = (jnp.dot(f, whbuf[0:64], preferred_element_type=jnp.float32)
                    + whbuf[64:65])

    @pl.when(i + _NBUF < nb)
    def _prefetch():
        start(i + _NBUF, slot)


def kernel(xi, h_A, h_B, E_val, I_val, L_val, q_val,
           W1, b1, W2, b2, Ww, bw, Wm, bm):
    B, H = h_A.shape
    D1 = W1.shape[1]
    D2 = W2.shape[1]

    scal = jnp.stack(
        [xi[:, 0], E_val[:, 0], I_val[:, 0], L_val[:, 0], q_val[:, 0]],
        axis=0)
    w1p = jnp.concatenate([
        W1[1:1 + H],
        W1[1 + H:1 + 2 * H],
        W1[0:1], W1[1 + 2 * H:],
        b1.reshape(1, D1),
    ], axis=0)
    w2p = jnp.concatenate([W2, b2.reshape(1, D2)], axis=0)
    whp = jnp.concatenate([
        jnp.concatenate([Ww, Wm], axis=1),
        jnp.concatenate([bw, bm]).reshape(1, 2),
    ], axis=0)

    grid = (B // _BS,)
    row = lambda i: (i, 0)

    out = pl.pallas_call(
        _mlp_kernel,
        grid=grid,
        in_specs=[
            pl.BlockSpec(memory_space=pl.ANY),
            pl.BlockSpec(memory_space=pl.ANY),
            pl.BlockSpec(memory_space=pl.ANY),
            pl.BlockSpec(memory_space=pl.ANY),
            pl.BlockSpec(memory_space=pl.ANY),
            pl.BlockSpec(memory_space=pl.ANY),
        ],
        out_specs=pl.BlockSpec((_BS, 2), row),
        out_shape=jax.ShapeDtypeStruct((B, 2), jnp.float32),
        scratch_shapes=[
            pltpu.VMEM((_NBUF, 5, _BS), jnp.float32),
            pltpu.VMEM((_NBUF, _BS, H), jnp.float32),
            pltpu.VMEM((_NBUF, _BS, H), jnp.float32),
            pltpu.VMEM((2 * H + 6, D1), jnp.float32),
            pltpu.VMEM((H + 1, D2), jnp.float32),
            pltpu.VMEM((D2 + 1, 2), jnp.float32),
            pltpu.SemaphoreType.DMA((3, _NBUF)),
            pltpu.SemaphoreType.DMA((3,)),
        ],
        compiler_params=pltpu.CompilerParams(
            dimension_semantics=("arbitrary",),
            vmem_limit_bytes=100 * 1024 * 1024),
    )(scal, h_A, h_B, w1p, w2p, whp)

    return (out[:, 0:1], out[:, 1:2])
